# Initial kernel scaffold; baseline (speedup 1.0000x reference)
import jax
import jax.numpy as jnp
from jax.experimental import pallas as pl


def kernel(node_deg, sample_pos):
    raise NotImplementedError("write your pallas kernel here")


# speedup vs baseline: 8.4575x; 8.4575x over previous
import functools

import jax
import jax.numpy as jnp
from jax import lax
from jax.experimental import pallas as pl
from jax.experimental.pallas import tpu as pltpu
from jax.experimental.pallas import tpu_sc as plsc

_N = 100000
_NSEG = 16
_NPAD = 102400
_C1 = _NPAD // 16
_C2 = _NPAD // 32
_L = 16

_mesh = plsc.VectorSubcoreMesh(core_axis_name="c", subcore_axis_name="s")


@functools.partial(
    pl.kernel,
    mesh=_mesh,
    compiler_params=pltpu.CompilerParams(needs_layout_passes=False),
    out_type=jax.ShapeDtypeStruct((_NPAD,), jnp.float32),
    scratch_types=[
        pltpu.VMEM((_C1,), jnp.float32),
        pltpu.VMEM((_C2,), jnp.float32),
        pltpu.VMEM((32,), jnp.int32),
        pltpu.VMEM((_L,), jnp.float32),
        pltpu.VMEM((16 * _L,), jnp.float32),
        pltpu.VMEM_SHARED((16 * _L,), jnp.float32),
    ],
)
def _filtration_kernel(deg_hbm, pos_hbm, out_hbm,
                       chunk_v, out_v, pos_v, stage_v, allp_v, shared):
    c = lax.axis_index("c")
    s = lax.axis_index("s")
    base1 = s * _C1

    pltpu.sync_copy(pos_hbm, pos_v)
    pltpu.sync_copy(deg_hbm.at[pl.ds(base1, _C1)], chunk_v)

    iota = lax.iota(jnp.int32, _L)
    ninf = jnp.full((_L,), -jnp.inf, dtype=jnp.float32)

    pos_lo_vec = pos_v[pl.ds(0, _L)]
    pos_hi_vec = pos_v[pl.ds(_L, _L)]
    pos = [pos_lo_vec[i] for i in range(_L)] + [pos_hi_vec[0]]

    pvec = ninf
    for seg in range(_NSEG):
        lo = jnp.maximum(pos[seg] - base1, 0)
        hi = jnp.minimum(pos[seg + 1] - base1, _C1)
        hi = jnp.maximum(hi, lo)
        start = lo // _L
        stop = (hi + (_L - 1)) // _L

        def body(j, acc, lo=lo, hi=hi):
            v = chunk_v[pl.ds(j * _L, _L)]
            idx = j * _L + iota
            m = (idx >= lo) & (idx < hi)
            return jnp.maximum(acc, jnp.where(m, v, ninf))

        acc = lax.fori_loop(start, stop, body, ninf)
        pvec = jnp.where(iota == seg, jnp.max(acc), pvec)

    stage_v[...] = pvec
    pltpu.sync_copy(stage_v, shared.at[pl.ds(s * _L, _L)])
    plsc.subcore_barrier()
    pltpu.sync_copy(shared, allp_v)
    gmax = ninf
    for r in range(16):
        gmax = jnp.maximum(gmax, allp_v[pl.ds(r * _L, _L)])
    inv = 1.0 / gmax

    off = c * _C2
    base2 = base1 + off
    for seg in range(_NSEG):
        lo = jnp.maximum(pos[seg] - base2, 0)
        hi = jnp.minimum(pos[seg + 1] - base2, _C2)
        hi = jnp.maximum(hi, lo)
        start = lo // _L
        stop = (hi + (_L - 1)) // _L
        scale = inv[seg]

        def body2(j, carry, lo=lo, hi=hi, scale=scale):
            v = chunk_v[pl.ds(off + j * _L, _L)]
            idx = j * _L + iota
            m = (idx >= lo) & (idx < hi)
            cur = out_v[pl.ds(j * _L, _L)]
            out_v[pl.ds(j * _L, _L)] = jnp.where(m, v * scale, cur)
            return carry

        lax.fori_loop(start, stop, body2, 0)

    pltpu.sync_copy(out_v, out_hbm.at[pl.ds(base2, _C2)])


def kernel(node_deg, sample_pos):
    deg = jnp.pad(node_deg.astype(jnp.float32), (0, _NPAD - _N))
    pos = jnp.pad(sample_pos.astype(jnp.int32), (0, 32 - 17),
                  constant_values=_N)
    out = _filtration_kernel(deg, pos)
    return out[:_N]


# speedup vs baseline: 8.8076x; 1.0414x over previous
import functools

import jax
import jax.numpy as jnp
from jax import lax
from jax.experimental import pallas as pl
from jax.experimental.pallas import tpu as pltpu
from jax.experimental.pallas import tpu_sc as plsc

_N = 100000
_NSEG = 16
_C1 = 6400
_C2 = 3200
_L = 16

_mesh = plsc.VectorSubcoreMesh(core_axis_name="c", subcore_axis_name="s")


@functools.partial(
    pl.kernel,
    mesh=_mesh,
    compiler_params=pltpu.CompilerParams(needs_layout_passes=False),
    out_type=jax.ShapeDtypeStruct((_N,), jnp.float32),
    scratch_types=[
        pltpu.VMEM((_C1,), jnp.float32),
        pltpu.VMEM((_C2,), jnp.float32),
        pltpu.VMEM((_L,), jnp.int32),
        pltpu.VMEM((_L,), jnp.float32),
        pltpu.VMEM((16 * _L,), jnp.float32),
        pltpu.VMEM_SHARED((16 * _L,), jnp.float32),
        pltpu.SemaphoreType.DMA,
    ],
)
def _filtration_kernel(deg_hbm, pos_hbm, out_hbm,
                       chunk_v, out_v, pos_v, stage_v, allp_v, shared, sem):
    c = lax.axis_index("c")
    s = lax.axis_index("s")
    base1 = s * _C1
    last1 = s == (_NSEG - 1)

    pos_cp = pltpu.async_copy(pos_hbm.at[pl.ds(0, _L)], pos_v, sem)

    @pl.when(jnp.logical_not(last1))
    def _():
        pltpu.sync_copy(deg_hbm.at[pl.ds(base1, _C1)], chunk_v)

    @pl.when(last1)
    def _():
        pltpu.sync_copy(deg_hbm.at[pl.ds(_N - 4000, 4000)],
                        chunk_v.at[pl.ds(0, 4000)])

    pos_cp.wait()

    iota = lax.iota(jnp.int32, _L)
    ninf = jnp.full((_L,), -jnp.inf, dtype=jnp.float32)

    pos_vec = pos_v[...]
    pos = [pos_vec[i] for i in range(_L)] + [jnp.int32(_N)]

    w1 = jnp.where(last1, 4000, _C1)
    jmax1 = w1 // _L - 1

    def masked_max(acc, j, lo, hi):
        v = chunk_v[pl.ds(j * _L, _L)]
        idx = j * _L + iota
        m = (idx >= lo) & (idx < hi)
        return jnp.maximum(acc, jnp.where(m, v, ninf))

    pvec = ninf
    for seg in range(_NSEG):
        lo = jnp.clip(pos[seg] - base1, 0, w1)
        hi = jnp.clip(pos[seg + 1] - base1, lo, w1)
        acc = masked_max(ninf, jnp.minimum(lo // _L, jmax1), lo, hi)
        acc = masked_max(acc, jnp.minimum(jnp.maximum(hi - 1, lo) // _L,
                                          jmax1), lo, hi)
        a = (lo + _L - 1) // _L
        b = jnp.maximum(a, hi // _L)

        def body(j, acc):
            return jnp.maximum(acc, chunk_v[pl.ds(j * _L, _L)])

        acc = plsc.parallel_loop(a, b, 1, unroll=4, carry=acc)(body)
        pvec = jnp.where(iota == seg, jnp.max(acc), pvec)

    stage_v[...] = pvec
    pltpu.sync_copy(stage_v, shared.at[pl.ds(s * _L, _L)])
    plsc.subcore_barrier()
    pltpu.sync_copy(shared, allp_v)
    gmax = ninf
    for r in range(16):
        gmax = jnp.maximum(gmax, allp_v[pl.ds(r * _L, _L)])
    inv = 1.0 / gmax

    off = c * _C2
    base2 = base1 + off
    last2 = last1 & (c == 1)
    w2 = jnp.where(last2, 800, _C2)
    jmax2 = w2 // _L - 1

    for seg in range(_NSEG):
        lo = jnp.clip(pos[seg] - base2, 0, w2)
        hi = jnp.clip(pos[seg + 1] - base2, lo, w2)
        scale = inv[seg]

        def edge(j, lo=lo, hi=hi, scale=scale):
            v = chunk_v[pl.ds(off + j * _L, _L)]
            idx = j * _L + iota
            m = (idx >= lo) & (idx < hi)
            cur = out_v[pl.ds(j * _L, _L)]
            out_v[pl.ds(j * _L, _L)] = jnp.where(m, v * scale, cur)

        edge(jnp.minimum(lo // _L, jmax2))
        edge(jnp.minimum(jnp.maximum(hi - 1, lo) // _L, jmax2))

        a = (lo + _L - 1) // _L
        b = jnp.maximum(a, hi // _L)

        def body2(j, scale=scale):
            out_v[pl.ds(j * _L, _L)] = (
                chunk_v[pl.ds(off + j * _L, _L)] * scale)

        plsc.parallel_loop(a, b, 1, unroll=4)(body2)

    @pl.when(jnp.logical_not(last2))
    def _():
        pltpu.sync_copy(out_v, out_hbm.at[pl.ds(base2, _C2)])

    @pl.when(last2)
    def _():
        pltpu.sync_copy(out_v.at[pl.ds(0, 800)],
                        out_hbm.at[pl.ds(_N - 800, 800)])


def kernel(node_deg, sample_pos):
    return _filtration_kernel(node_deg.astype(jnp.float32),
                              sample_pos.astype(jnp.int32))
